# baseline (speedup 1.0000x reference)
import functools
import jax
import jax.numpy as jnp
from jax import lax
from jax.experimental import pallas as pl
from jax.experimental.pallas import tpu as pltpu
from jax.experimental.pallas import tpu_sc as plsc

_E = 16
_D = 768
_FF = 384
_T = 2048
_BT = 256
_MAXBLK = 31
_MAXROWS = _MAXBLK * _BT
_DW = _D + 128


def _mm_t(a, b):
    return jax.lax.dot_general(
        a, b, (((1,), (1,)), ((), ())), preferred_element_type=jnp.float32
    )


def _silu(g):
    return g * (1.0 / (1.0 + jnp.exp(-g)))


def _route_body(x_ref, gw_ref, sgu_ref, sd_ref,
                s1_ref, s2_ref, eidb_ref, sh_ref, xa_ref):
    x = x_ref[...]
    logits = _mm_t(x, gw_ref[...])
    iota = jax.lax.broadcasted_iota(jnp.int32, (_T, _E), 1)
    m1 = jnp.max(logits, axis=-1, keepdims=True)
    is1 = logits == m1
    j1 = jnp.min(jnp.where(is1, iota, _E), axis=-1, keepdims=True)
    oh1 = (iota == j1).astype(jnp.float32)
    rest = jnp.where(iota == j1, -jnp.inf, logits)
    m2 = jnp.max(rest, axis=-1, keepdims=True)
    is2 = rest == m2
    j2 = jnp.min(jnp.where(is2, iota, _E), axis=-1, keepdims=True)
    oh2 = (iota == j2).astype(jnp.float32)
    e2 = jnp.exp(m2 - m1)
    denom = 1.0 + e2
    w1c = 1.0 / denom
    w2c = e2 / denom
    xa_ref[...] = jnp.concatenate(
        [x, w1c, w2c, jnp.zeros((_T, 126), jnp.float32)], axis=1)

    ti = jax.lax.broadcasted_iota(jnp.int32, (_T, _T), 0)
    tj = jax.lax.broadcasted_iota(jnp.int32, (_T, _T), 1)
    tril = (ti > tj).astype(jnp.bfloat16)
    oh12 = jnp.concatenate([oh1, oh2], axis=1).astype(jnp.bfloat16)
    pre12 = jax.lax.dot_general(
        tril, oh12, (((1,), (0,)), ((), ())),
        preferred_element_type=jnp.float32)
    pre1 = pre12[:, :_E]
    pre2 = pre12[:, _E:]
    cnt1 = jnp.sum(oh1, axis=0, keepdims=True)
    cnt2 = jnp.sum(oh2, axis=0, keepdims=True)
    cnt = cnt1 + cnt2
    blocks = jnp.ceil(cnt * (1.0 / _BT))
    ei = jax.lax.broadcasted_iota(jnp.int32, (_E, _E), 0)
    ej = jax.lax.broadcasted_iota(jnp.int32, (_E, _E), 1)
    trile = (ei <= ej).astype(jnp.float32)
    cumblk = jax.lax.dot_general(
        blocks, trile, (((1,), (0,)), ((), ())),
        preferred_element_type=jnp.float32)
    base = (cumblk - blocks) * float(_BT)

    s1 = jnp.sum(oh1 * (base + pre1), axis=-1, keepdims=True)
    s2 = jnp.sum(oh2 * (base + cnt1 + pre2), axis=-1, keepdims=True)
    s1_ref[...] = s1.astype(jnp.int32)
    s2_ref[...] = s2.astype(jnp.int32)

    bi = jax.lax.broadcasted_iota(jnp.int32, (32, _E), 0).astype(jnp.float32)
    eidb = jnp.sum((bi >= cumblk).astype(jnp.int32), axis=-1, keepdims=True)
    eidb_ref[...] = jnp.minimum(eidb, _E - 1)

    sgu = _mm_t(x, sgu_ref[...])
    sact = _silu(sgu[:, :_FF]) * sgu[:, _FF:]
    sh_ref[...] = _mm_t(sact, sd_ref[...])


def _route(x, gate_w, shared_gate_up, shared_down):
    return pl.pallas_call(
        _route_body,
        grid=(1,),
        in_specs=[
            pl.BlockSpec((_T, _D), lambda i: (0, 0)),
            pl.BlockSpec((_E, _D), lambda i: (0, 0)),
            pl.BlockSpec((2 * _FF, _D), lambda i: (0, 0)),
            pl.BlockSpec((_D, _FF), lambda i: (0, 0)),
        ],
        out_specs=[
            pl.BlockSpec((_T, 1), lambda i: (0, 0)),
            pl.BlockSpec((_T, 1), lambda i: (0, 0)),
            pl.BlockSpec((32, 1), lambda i: (0, 0)),
            pl.BlockSpec((_T, _D), lambda i: (0, 0)),
            pl.BlockSpec((_T, _DW), lambda i: (0, 0)),
        ],
        out_shape=[
            jax.ShapeDtypeStruct((_T, 1), jnp.int32),
            jax.ShapeDtypeStruct((_T, 1), jnp.int32),
            jax.ShapeDtypeStruct((32, 1), jnp.int32),
            jax.ShapeDtypeStruct((_T, _D), jnp.float32),
            jax.ShapeDtypeStruct((_T, _DW), jnp.float32),
        ],
    )(x, gate_w, shared_gate_up, shared_down)


_mesh = plsc.VectorSubcoreMesh(core_axis_name="c", subcore_axis_name="s")
_TPW = _T // 32


@functools.partial(
    pl.kernel,
    out_type=jax.ShapeDtypeStruct((_MAXROWS, _DW), jnp.float32),
    mesh=_mesh,
    scratch_types=[
        pltpu.VMEM((_TPW,), jnp.int32),
        pltpu.VMEM((_TPW,), jnp.int32),
        pltpu.VMEM((_TPW, _DW), jnp.float32),
        pltpu.SemaphoreType.DMA,
    ],
)
def _dispatch(xa_hbm, s1_hbm, s2_hbm, xs_hbm, s1_v, s2_v, rows_v, sem):
    wid = lax.axis_index("s") * 2 + lax.axis_index("c")
    t0 = wid * _TPW
    pltpu.sync_copy(s1_hbm.at[pl.ds(t0, _TPW)], s1_v)
    pltpu.sync_copy(s2_hbm.at[pl.ds(t0, _TPW)], s2_v)
    pltpu.sync_copy(xa_hbm.at[pl.ds(t0, _TPW)], rows_v)
    c1 = pltpu.async_copy(rows_v, xs_hbm.at[s1_v], sem)
    c1.wait()
    for l in range(_TPW):
        row = rows_v.at[l]
        v16 = row[pl.ds(_D, 16)]
        row[pl.ds(_D, 16)] = jnp.zeros((16,), jnp.float32) + v16[1]
    c2 = pltpu.async_copy(rows_v, xs_hbm.at[s2_v], sem)
    c2.wait()


def _group_body(eidb_ref, xs_ref, wgu_ref, wd_ref, ys_ref):
    xsw = xs_ref[...]
    xs = xsw[:, :_D]
    wv = xsw[:, _D:_D + 1]
    gu = _mm_t(xs, wgu_ref[0])
    act = _silu(gu[:, :_FF]) * gu[:, _FF:]
    ys_ref[...] = _mm_t(act, wd_ref[0]) * wv


def _grouped(eidb, xs, expert_gate_up, expert_down):
    grid_spec = pltpu.PrefetchScalarGridSpec(
        num_scalar_prefetch=1,
        grid=(_MAXBLK,),
        in_specs=[
            pl.BlockSpec((_BT, _DW), lambda b, eidb: (b, 0)),
            pl.BlockSpec((1, 2 * _FF, _D), lambda b, eidb: (eidb[b], 0, 0)),
            pl.BlockSpec((1, _D, _FF), lambda b, eidb: (eidb[b], 0, 0)),
        ],
        out_specs=pl.BlockSpec((_BT, _D), lambda b, eidb: (b, 0)),
    )
    return pl.pallas_call(
        _group_body,
        grid_spec=grid_spec,
        out_shape=jax.ShapeDtypeStruct((_MAXROWS, _D), jnp.float32),
        compiler_params=pltpu.CompilerParams(
            dimension_semantics=("arbitrary",),
        ),
    )(eidb, xs, expert_gate_up, expert_down)


@functools.partial(
    pl.kernel,
    out_type=jax.ShapeDtypeStruct((_T, _D), jnp.float32),
    mesh=_mesh,
    scratch_types=[
        pltpu.VMEM((_TPW,), jnp.int32),
        pltpu.VMEM((_TPW,), jnp.int32),
        pltpu.VMEM((_TPW, _D), jnp.float32),
        pltpu.VMEM((_TPW, _D), jnp.float32),
        pltpu.SemaphoreType.DMA,
    ],
)
def _combine(sh_hbm, ys_hbm, s1_hbm, s2_hbm, out_hbm,
             s1_v, s2_v, acc_v, buf_v, sem):
    wid = lax.axis_index("s") * 2 + lax.axis_index("c")
    t0 = wid * _TPW
    pltpu.sync_copy(s1_hbm.at[pl.ds(t0, _TPW)], s1_v)
    pltpu.sync_copy(s2_hbm.at[pl.ds(t0, _TPW)], s2_v)
    pltpu.sync_copy(sh_hbm.at[pl.ds(t0, _TPW)], acc_v)

    def accumulate():
        def tb(t, _):
            arow = acc_v.at[t]
            brow = buf_v.at[t]
            for j in range(_D // 16):
                sl = pl.ds(j * 16, 16)
                arow[sl] = arow[sl] + brow[sl]
            return 0
        lax.fori_loop(0, _TPW, tb, 0)

    pltpu.async_copy(ys_hbm.at[s1_v], buf_v, sem).wait()
    accumulate()
    pltpu.async_copy(ys_hbm.at[s2_v], buf_v, sem).wait()
    accumulate()
    pltpu.sync_copy(acc_v, out_hbm.at[pl.ds(t0, _TPW)])


def kernel(hidden_states, gate_w, expert_gate_up, expert_down,
           shared_gate_up, shared_down):
    s1, s2, eidb, shared_out, x_aug = _route(
        hidden_states, gate_w, shared_gate_up, shared_down)
    s1 = s1.reshape(_T)
    s2 = s2.reshape(_T)
    eidb = eidb.reshape(32)
    xs = _dispatch(x_aug, s1, s2)
    ys = _grouped(eidb, xs, expert_gate_up, expert_down)
    return _combine(shared_out, ys, s1, s2)


# speedup vs baseline: 1.0746x; 1.0746x over previous
import functools
import jax
import jax.numpy as jnp
from jax import lax
from jax.experimental import pallas as pl
from jax.experimental.pallas import tpu as pltpu
from jax.experimental.pallas import tpu_sc as plsc

_E = 16
_D = 768
_FF = 384
_T = 2048
_BT = 256
_MAXBLK = 31
_MAXROWS = _MAXBLK * _BT
_DW = _D + 128


def _mm_t(a, b):
    return jax.lax.dot_general(
        a, b, (((1,), (1,)), ((), ())), preferred_element_type=jnp.float32
    )


def _silu(g):
    return g * (1.0 / (1.0 + jnp.exp(-g)))


def _route_body(x_ref, gw_ref, sgu_ref, sd_ref,
                s1_ref, s2_ref, eidb_ref, sh_ref, xa_ref):
    x = x_ref[...]
    logits = _mm_t(x, gw_ref[...])
    iota = jax.lax.broadcasted_iota(jnp.int32, (_T, _E), 1)
    m1 = jnp.max(logits, axis=-1, keepdims=True)
    is1 = logits == m1
    j1 = jnp.min(jnp.where(is1, iota, _E), axis=-1, keepdims=True)
    oh1 = (iota == j1).astype(jnp.float32)
    rest = jnp.where(iota == j1, -jnp.inf, logits)
    m2 = jnp.max(rest, axis=-1, keepdims=True)
    is2 = rest == m2
    j2 = jnp.min(jnp.where(is2, iota, _E), axis=-1, keepdims=True)
    oh2 = (iota == j2).astype(jnp.float32)
    e2 = jnp.exp(m2 - m1)
    denom = 1.0 + e2
    w1c = 1.0 / denom
    w2c = e2 / denom
    xa_ref[...] = jnp.concatenate(
        [x, w1c, w2c, jnp.zeros((_T, 126), jnp.float32)], axis=1)

    ti = jax.lax.broadcasted_iota(jnp.int32, (_T, _T), 0)
    tj = jax.lax.broadcasted_iota(jnp.int32, (_T, _T), 1)
    tril = (ti > tj).astype(jnp.bfloat16)
    oh12 = jnp.concatenate([oh1, oh2], axis=1).astype(jnp.bfloat16)
    pre12 = jax.lax.dot_general(
        tril, oh12, (((1,), (0,)), ((), ())),
        preferred_element_type=jnp.float32)
    pre1 = pre12[:, :_E]
    pre2 = pre12[:, _E:]
    cnt1 = jnp.sum(oh1, axis=0, keepdims=True)
    cnt2 = jnp.sum(oh2, axis=0, keepdims=True)
    cnt = cnt1 + cnt2
    blocks = jnp.ceil(cnt * (1.0 / _BT))
    ei = jax.lax.broadcasted_iota(jnp.int32, (_E, _E), 0)
    ej = jax.lax.broadcasted_iota(jnp.int32, (_E, _E), 1)
    trile = (ei <= ej).astype(jnp.float32)
    cumblk = jax.lax.dot_general(
        blocks, trile, (((1,), (0,)), ((), ())),
        preferred_element_type=jnp.float32)
    base = (cumblk - blocks) * float(_BT)

    s1 = jnp.sum(oh1 * (base + pre1), axis=-1, keepdims=True)
    s2 = jnp.sum(oh2 * (base + cnt1 + pre2), axis=-1, keepdims=True)
    s1_ref[...] = s1.astype(jnp.int32)
    s2_ref[...] = s2.astype(jnp.int32)

    bi = jax.lax.broadcasted_iota(jnp.int32, (32, _E), 0).astype(jnp.float32)
    eidb = jnp.sum((bi >= cumblk).astype(jnp.int32), axis=-1, keepdims=True)
    eidb = jnp.minimum(eidb, _E - 1)
    bidx = jax.lax.broadcasted_iota(jnp.int32, (32, 1), 0)
    nblk = jnp.sum(cumblk[:, _E - 1:_E]).astype(jnp.int32)
    eidb_ref[...] = jnp.where(bidx == 31, nblk, eidb)

    sgu = _mm_t(x, sgu_ref[...])
    sact = _silu(sgu[:, :_FF]) * sgu[:, _FF:]
    sh_ref[...] = _mm_t(sact, sd_ref[...])


def _route(x, gate_w, shared_gate_up, shared_down):
    return pl.pallas_call(
        _route_body,
        grid=(1,),
        in_specs=[
            pl.BlockSpec((_T, _D), lambda i: (0, 0)),
            pl.BlockSpec((_E, _D), lambda i: (0, 0)),
            pl.BlockSpec((2 * _FF, _D), lambda i: (0, 0)),
            pl.BlockSpec((_D, _FF), lambda i: (0, 0)),
        ],
        out_specs=[
            pl.BlockSpec((_T, 1), lambda i: (0, 0)),
            pl.BlockSpec((_T, 1), lambda i: (0, 0)),
            pl.BlockSpec((32, 1), lambda i: (0, 0)),
            pl.BlockSpec((_T, _D), lambda i: (0, 0)),
            pl.BlockSpec((_T, _DW), lambda i: (0, 0)),
        ],
        out_shape=[
            jax.ShapeDtypeStruct((_T, 1), jnp.int32),
            jax.ShapeDtypeStruct((_T, 1), jnp.int32),
            jax.ShapeDtypeStruct((32, 1), jnp.int32),
            jax.ShapeDtypeStruct((_T, _D), jnp.float32),
            jax.ShapeDtypeStruct((_T, _DW), jnp.float32),
        ],
    )(x, gate_w, shared_gate_up, shared_down)


_mesh = plsc.VectorSubcoreMesh(core_axis_name="c", subcore_axis_name="s")
_TPW = _T // 32


@functools.partial(
    pl.kernel,
    out_type=jax.ShapeDtypeStruct((_MAXROWS, _DW), jnp.float32),
    mesh=_mesh,
    scratch_types=[
        pltpu.VMEM((_TPW,), jnp.int32),
        pltpu.VMEM((_TPW,), jnp.int32),
        pltpu.VMEM((_TPW, _DW), jnp.float32),
        pltpu.SemaphoreType.DMA,
    ],
)
def _dispatch(xa_hbm, s1_hbm, s2_hbm, xs_hbm, s1_v, s2_v, rows_v, sem):
    wid = lax.axis_index("s") * 2 + lax.axis_index("c")
    t0 = wid * _TPW
    pltpu.sync_copy(s1_hbm.at[pl.ds(t0, _TPW)], s1_v)
    pltpu.sync_copy(s2_hbm.at[pl.ds(t0, _TPW)], s2_v)
    pltpu.sync_copy(xa_hbm.at[pl.ds(t0, _TPW)], rows_v)
    c1 = pltpu.async_copy(rows_v, xs_hbm.at[s1_v], sem)
    c1.wait()
    for l in range(_TPW):
        row = rows_v.at[l]
        v16 = row[pl.ds(_D, 16)]
        row[pl.ds(_D, 16)] = jnp.zeros((16,), jnp.float32) + v16[1]
    c2 = pltpu.async_copy(rows_v, xs_hbm.at[s2_v], sem)
    c2.wait()


def _group_body(eidb_ref, xs_ref, wgu_ref, wd_ref, ys_ref):
    b = pl.program_id(0)

    @pl.when(b < eidb_ref[31])
    def _():
        xsw = xs_ref[...]
        xs = xsw[:, :_D]
        wv = xsw[:, _D:_D + 1]
        gu = _mm_t(xs, wgu_ref[0])
        act = _silu(gu[:, :_FF]) * gu[:, _FF:]
        ys_ref[...] = _mm_t(act, wd_ref[0]) * wv


def _grouped(eidb, xs, expert_gate_up, expert_down):
    grid_spec = pltpu.PrefetchScalarGridSpec(
        num_scalar_prefetch=1,
        grid=(_MAXBLK,),
        in_specs=[
            pl.BlockSpec(
                (_BT, _DW),
                lambda b, eidb: (jnp.minimum(b, eidb[31] - 1), 0)),
            pl.BlockSpec((1, 2 * _FF, _D), lambda b, eidb: (eidb[b], 0, 0)),
            pl.BlockSpec((1, _D, _FF), lambda b, eidb: (eidb[b], 0, 0)),
        ],
        out_specs=pl.BlockSpec(
            (_BT, _D),
            lambda b, eidb: (jnp.minimum(b, eidb[31] - 1), 0)),
    )
    return pl.pallas_call(
        _group_body,
        grid_spec=grid_spec,
        out_shape=jax.ShapeDtypeStruct((_MAXROWS, _D), jnp.float32),
        compiler_params=pltpu.CompilerParams(
            dimension_semantics=("arbitrary",),
        ),
    )(eidb, xs, expert_gate_up, expert_down)


@functools.partial(
    pl.kernel,
    out_type=jax.ShapeDtypeStruct((_T, _D), jnp.float32),
    mesh=_mesh,
    scratch_types=[
        pltpu.VMEM((_TPW,), jnp.int32),
        pltpu.VMEM((_TPW,), jnp.int32),
        pltpu.VMEM((_TPW // 2, _D), jnp.float32),
        pltpu.VMEM((_TPW // 2, _D), jnp.float32),
        pltpu.VMEM((_TPW // 2, _D), jnp.float32),
        pltpu.SemaphoreType.DMA,
        pltpu.SemaphoreType.DMA,
        pltpu.SemaphoreType.DMA,
    ],
)
def _combine(sh_hbm, ys_hbm, s1_hbm, s2_hbm, out_hbm,
             s1_v, s2_v, acc_v, b1_v, b2_v, sem0, sem1, sem2):
    wid = lax.axis_index("s") * 2 + lax.axis_index("c")
    t0 = wid * _TPW
    half = _TPW // 2
    pltpu.sync_copy(s1_hbm.at[pl.ds(t0, _TPW)], s1_v)
    pltpu.sync_copy(s2_hbm.at[pl.ds(t0, _TPW)], s2_v)
    for h in range(2):
        th = t0 + h * half
        c0 = pltpu.async_copy(sh_hbm.at[pl.ds(th, half)], acc_v, sem0)
        c1 = pltpu.async_copy(ys_hbm.at[s1_v.at[pl.ds(h * half, half)]],
                              b1_v, sem1)
        c2 = pltpu.async_copy(ys_hbm.at[s2_v.at[pl.ds(h * half, half)]],
                              b2_v, sem2)
        c0.wait()
        c1.wait()
        c2.wait()

        def tb(t, _):
            arow = acc_v.at[t]
            brow = b1_v.at[t]
            crow = b2_v.at[t]
            for j in range(_D // 16):
                sl = pl.ds(j * 16, 16)
                arow[sl] = arow[sl] + (brow[sl] + crow[sl])
            return 0
        lax.fori_loop(0, half, tb, 0)
        pltpu.sync_copy(acc_v, out_hbm.at[pl.ds(th, half)])


def kernel(hidden_states, gate_w, expert_gate_up, expert_down,
           shared_gate_up, shared_down):
    s1, s2, eidb, shared_out, x_aug = _route(
        hidden_states, gate_w, shared_gate_up, shared_down)
    s1 = s1.reshape(_T)
    s2 = s2.reshape(_T)
    eidb = eidb.reshape(32)
    xs = _dispatch(x_aug, s1, s2)
    ys = _grouped(eidb, xs, expert_gate_up, expert_down)
    return _combine(shared_out, ys, s1, s2)
